# baseline (device time: 180120 ns/iter reference)
import jax
import jax.numpy as jnp
from jax import lax
from jax.experimental import pallas as pl
from jax.experimental.pallas import tpu as pltpu

N_DEV = 8


def kernel(x, w_mat):
    m_total, k_local = x.shape
    k2, n = w_mat.shape
    m_per = m_total // N_DEV

    def body(x_ref, w_ref, out_ref, acc_ref, recv_ref, send_sems, recv_sems):
        my = lax.axis_index("i")
        left = (my - 1) % N_DEV
        right = (my + 1) % N_DEV

        barrier_sem = pltpu.get_barrier_semaphore()
        for nbr in (left, right):
            pl.semaphore_signal(
                barrier_sem, inc=1,
                device_id=(nbr,), device_id_type=pl.DeviceIdType.MESH,
            )
        pl.semaphore_wait(barrier_sem, 2)

        w = w_ref[...].astype(jnp.bfloat16)

        def contrib(c):
            xa = x_ref[pl.ds(c * m_per, m_per), :].astype(jnp.bfloat16)
            return jnp.dot(xa, w, preferred_element_type=jnp.float32)

        acc_ref[...] = contrib((my - 1) % N_DEV)

        for h in range(N_DEV - 1):
            rdma = pltpu.make_async_remote_copy(
                src_ref=acc_ref,
                dst_ref=recv_ref.at[h],
                send_sem=send_sems.at[h],
                recv_sem=recv_sems.at[h],
                device_id=(right,),
                device_id_type=pl.DeviceIdType.MESH,
            )
            rdma.start()
            part = contrib((my - (h + 2)) % N_DEV)
            rdma.wait_send()
            rdma.wait_recv()
            if h < N_DEV - 2:
                acc_ref[...] = recv_ref[h] + part
            else:
                out_ref[...] = recv_ref[h] + part

    return pl.pallas_call(
        body,
        out_shape=jax.ShapeDtypeStruct((m_per, n), jnp.float32),
        in_specs=[
            pl.BlockSpec(memory_space=pltpu.VMEM),
            pl.BlockSpec(memory_space=pltpu.VMEM),
        ],
        out_specs=pl.BlockSpec(memory_space=pltpu.VMEM),
        scratch_shapes=[
            pltpu.VMEM((m_per, n), jnp.float32),
            pltpu.VMEM((N_DEV - 1, m_per, n), jnp.float32),
            pltpu.SemaphoreType.DMA((N_DEV - 1,)),
            pltpu.SemaphoreType.DMA((N_DEV - 1,)),
        ],
        compiler_params=pltpu.CompilerParams(collective_id=0),
    )(x, w_mat)


# device time: 76011 ns/iter; 2.3697x vs baseline; 2.3697x over previous
import jax
import jax.numpy as jnp
from jax import lax
from jax.experimental import pallas as pl
from jax.experimental.pallas import tpu as pltpu

N_DEV = 8


def kernel(x, w_mat):
    m_total, k_local = x.shape
    k2, n = w_mat.shape
    m_per = m_total // N_DEV

    def body(x_ref, w_ref, out_ref, send_ref, recv_ref, send_sems, recv_sems):
        my = lax.axis_index("i")

        barrier_sem = pltpu.get_barrier_semaphore()
        for t in range(1, N_DEV):
            pl.semaphore_signal(
                barrier_sem, inc=1,
                device_id=((my + t) % N_DEV,),
                device_id_type=pl.DeviceIdType.MESH,
            )
        pl.semaphore_wait(barrier_sem, N_DEV - 1)

        w = w_ref[...].astype(jnp.bfloat16)

        def contrib(c):
            xa = x_ref[pl.ds(c * m_per, m_per), :].astype(jnp.bfloat16)
            return jnp.dot(xa, w, preferred_element_type=jnp.float32)

        rdmas = []
        for t in range(N_DEV - 1):
            dest = (my + 1 + t) % N_DEV
            send_ref[t, ...] = contrib(dest).astype(jnp.bfloat16)
            rdma = pltpu.make_async_remote_copy(
                src_ref=send_ref.at[t],
                dst_ref=recv_ref.at[N_DEV - 2 - t],
                send_sem=send_sems.at[t],
                recv_sem=recv_sems.at[N_DEV - 2 - t],
                device_id=(dest,),
                device_id_type=pl.DeviceIdType.MESH,
            )
            rdma.start()
            rdmas.append(rdma)

        out_ref[...] = contrib(my)

        for j in range(N_DEV - 2, -1, -1):
            rdmas[N_DEV - 2 - j].wait_recv()
            out_ref[...] = out_ref[...] + recv_ref[j, ...].astype(jnp.float32)

        for t in range(N_DEV - 1):
            rdmas[t].wait_send()

    return pl.pallas_call(
        body,
        out_shape=jax.ShapeDtypeStruct((m_per, n), jnp.float32),
        in_specs=[
            pl.BlockSpec(memory_space=pltpu.VMEM),
            pl.BlockSpec(memory_space=pltpu.VMEM),
        ],
        out_specs=pl.BlockSpec(memory_space=pltpu.VMEM),
        scratch_shapes=[
            pltpu.VMEM((N_DEV - 1, m_per, n), jnp.bfloat16),
            pltpu.VMEM((N_DEV - 1, m_per, n), jnp.bfloat16),
            pltpu.SemaphoreType.DMA((N_DEV - 1,)),
            pltpu.SemaphoreType.DMA((N_DEV - 1,)),
        ],
        compiler_params=pltpu.CompilerParams(collective_id=0),
    )(x, w_mat)


# device time: 60839 ns/iter; 2.9606x vs baseline; 1.2494x over previous
import jax
import jax.numpy as jnp
from jax import lax
from jax.experimental import pallas as pl
from jax.experimental.pallas import tpu as pltpu

N_DEV = 8

X = (0, 1, 1, 0, 0, 1, 1, 0)
Y = (0, 0, 1, 1, 0, 0, 1, 1)
Z = (0, 0, 0, 0, 1, 1, 1, 1)
COORDS = (X, Y, Z)
MASK = (1, 3, 4)

ORDERS = (
    (0, 1, 2), (0, 1, 2), (1, 2, 0), (2, 0, 1),
    (0, 1, 2), (0, 1, 2), (1, 2, 0), (2, 0, 1),
)


def kernel(x, w_mat):
    m_total, k_local = x.shape
    k2, n = w_mat.shape
    m_per = m_total // N_DEV

    def body(x_ref, w_ref, out_ref, send_ref, recv_ref, send_sems, recv_sems):
        my = lax.axis_index("i")
        px = (my ^ (my >> 1)) & 1
        py = (my >> 1) & 1
        pz = (my >> 2) & 1
        pc = (px, py, pz)

        barrier_sem = pltpu.get_barrier_semaphore()
        for mask in MASK:
            pl.semaphore_signal(
                barrier_sem, inc=1,
                device_id=(my ^ mask,), device_id_type=pl.DeviceIdType.MESH,
            )
        pl.semaphore_wait(barrier_sem, 3)

        w = w_ref[...].astype(jnp.bfloat16)

        def send_rdma(q, step, axis):
            return pltpu.make_async_remote_copy(
                src_ref=send_ref.at[q],
                dst_ref=recv_ref.at[q, step],
                send_sem=send_sems.at[q],
                recv_sem=recv_sems.at[q, step],
                device_id=(my ^ MASK[axis],),
                device_id_type=pl.DeviceIdType.MESH,
            )

        def wait_recv(q, step):
            pltpu.make_async_remote_copy(
                src_ref=send_ref.at[q],
                dst_ref=recv_ref.at[q, step],
                send_sem=send_sems.at[q],
                recv_sem=recv_sems.at[q, step],
                device_id=(my,),
                device_id_type=pl.DeviceIdType.MESH,
            ).wait_recv()

        def add_into_send(q, *recv_steps):
            acc = send_ref[q, ...].astype(jnp.float32)
            for s in recv_steps:
                acc = acc + recv_ref[q, s, ...].astype(jnp.float32)
            send_ref[q, ...] = acc.astype(jnp.bfloat16)

        def cdiff(q):
            d1, d2, d3 = ORDERS[q]
            return (
                pc[d1] != COORDS[d1][q],
                pc[d2] != COORDS[d2][q],
                pc[d3] != COORDS[d3][q],
            )

        for q in range(N_DEV):
            xa = x_ref[q * m_per:(q + 1) * m_per, :].astype(jnp.bfloat16)
            send_ref[q, ...] = jnp.dot(
                xa, w, preferred_element_type=jnp.float32
            ).astype(jnp.bfloat16)
            c1, _, _ = cdiff(q)

            @pl.when(c1)
            def _(q=q):
                send_rdma(q, 0, ORDERS[q][0]).start()

        for q in range(N_DEV):
            c1, c2, _ = cdiff(q)

            @pl.when(jnp.logical_and(~c1, c2))
            def _(q=q):
                wait_recv(q, 0)
                add_into_send(q, 0)
                send_rdma(q, 1, ORDERS[q][1]).start()

        for q in range(N_DEV):
            c1, c2, c3 = cdiff(q)

            @pl.when(jnp.logical_and(~c1, jnp.logical_and(~c2, c3)))
            def _(q=q):
                wait_recv(q, 0)
                wait_recv(q, 1)
                add_into_send(q, 0, 1)
                send_rdma(q, 2, ORDERS[q][2]).start()

        for q in range(N_DEV):
            @pl.when(my == q)
            def _(q=q):
                wait_recv(q, 0)
                wait_recv(q, 1)
                wait_recv(q, 2)
                out_ref[...] = (
                    send_ref[q, ...].astype(jnp.float32)
                    + recv_ref[q, 0, ...].astype(jnp.float32)
                    + recv_ref[q, 1, ...].astype(jnp.float32)
                    + recv_ref[q, 2, ...].astype(jnp.float32)
                )

        for q in range(N_DEV):
            @pl.when(my != q)
            def _(q=q):
                pltpu.make_async_remote_copy(
                    src_ref=send_ref.at[q],
                    dst_ref=recv_ref.at[q, 0],
                    send_sem=send_sems.at[q],
                    recv_sem=recv_sems.at[q, 0],
                    device_id=(my,),
                    device_id_type=pl.DeviceIdType.MESH,
                ).wait_send()

    return pl.pallas_call(
        body,
        out_shape=jax.ShapeDtypeStruct((m_per, n), jnp.float32),
        in_specs=[
            pl.BlockSpec(memory_space=pltpu.VMEM),
            pl.BlockSpec(memory_space=pltpu.VMEM),
        ],
        out_specs=pl.BlockSpec(memory_space=pltpu.VMEM),
        scratch_shapes=[
            pltpu.VMEM((N_DEV, m_per, n), jnp.bfloat16),
            pltpu.VMEM((N_DEV, 3, m_per, n), jnp.bfloat16),
            pltpu.SemaphoreType.DMA((N_DEV,)),
            pltpu.SemaphoreType.DMA((N_DEV, 3)),
        ],
        compiler_params=pltpu.CompilerParams(
            collective_id=0, vmem_limit_bytes=100 * 1024 * 1024
        ),
    )(x, w_mat)
